# baseline (speedup 1.0000x reference)
import functools

import jax
import jax.numpy as jnp
from jax import lax
from jax.experimental import pallas as pl
from jax.experimental.pallas import tpu as pltpu
from jax.experimental.pallas import tpu_sc as plsc

HID = 128
N_NODES = 10000
N_MAIN = 8000
N_SUPE = N_NODES - N_MAIN
N_EDGES = 320000
ROW_BLK = 1000

NC = 2
NS = 16
NW = NC * NS
EB = 80


def _node_stage_body(x_ref, wam_ref, wamb_ref, was_ref, wasb_ref,
                     cmsw_ref, cmsb_ref, cmaw_ref, cmmw_ref, cmmb_ref,
                     cssw_ref, cssb_ref, csaw_ref, csmw_ref, csmb_ref,
                     xw_ref, u1_ref, m1_ref, u2_ref, m2_ref):
    i = pl.program_id(0)
    is_main = i * ROW_BLK < N_MAIN
    xb = x_ref[...]
    w = jnp.where(is_main, wam_ref[...], was_ref[...])
    b = jnp.where(is_main, wamb_ref[...], wasb_ref[...])
    xw = jnp.tanh(jnp.dot(xb, w.T, preferred_element_type=jnp.float32) + b)
    xw_ref[...] = xw
    u1_ref[...] = (jnp.dot(xw, cmsw_ref[...].T,
                           preferred_element_type=jnp.float32)
                   + cmsb_ref[...]) * cmaw_ref[...]
    m1_ref[...] = jnp.dot(xw, cmmw_ref[...].T,
                          preferred_element_type=jnp.float32) + cmmb_ref[...]
    u2_ref[...] = (jnp.dot(xw, cssw_ref[...].T,
                           preferred_element_type=jnp.float32)
                   + cssb_ref[...]) * csaw_ref[...]
    m2_ref[...] = jnp.dot(xw, csmw_ref[...].T,
                          preferred_element_type=jnp.float32) + csmb_ref[...]


def _node_stage(x, wam_w, wam_b, was_w, was_b,
                cm_supe_w, cm_supe_b, cm_attn_w, cm_main_w, cm_main_b,
                cs_supe_w, cs_supe_b, cs_attn_w, cs_main_w, cs_main_b):
    n = x.shape[0]
    row_spec = pl.BlockSpec((ROW_BLK, HID), lambda i: (i, 0))
    full_spec = pl.BlockSpec((HID, HID), lambda i: (0, 0))
    vec_spec = pl.BlockSpec((1, HID), lambda i: (0, 0))
    out_sds = jax.ShapeDtypeStruct((n, HID), jnp.float32)
    return pl.pallas_call(
        _node_stage_body,
        grid=(n // ROW_BLK,),
        in_specs=[row_spec,
                  full_spec, vec_spec, full_spec, vec_spec,
                  full_spec, vec_spec, vec_spec, full_spec, vec_spec,
                  full_spec, vec_spec, vec_spec, full_spec, vec_spec],
        out_specs=[row_spec] * 5,
        out_shape=[out_sds] * 5,
    )(x, wam_w, wam_b.reshape(1, HID), was_w, was_b.reshape(1, HID),
      cm_supe_w, cm_supe_b.reshape(1, HID), cm_attn_w, cm_main_w,
      cm_main_b.reshape(1, HID),
      cs_supe_w, cs_supe_b.reshape(1, HID), cs_attn_w, cs_main_w,
      cs_main_b.reshape(1, HID))


E_PER_TILE = N_EDGES // NW
N_BATCH = E_PER_TILE // EB


def _make_edge_logits():
    mesh = plsc.VectorSubcoreMesh(core_axis_name="c", subcore_axis_name="s",
                                  num_cores=NC, num_subcores=NS)

    @functools.partial(
        pl.kernel,
        out_type=[jax.ShapeDtypeStruct((N_EDGES, 16), jnp.float32),
                  jax.ShapeDtypeStruct((N_EDGES, 16), jnp.float32)],
        mesh=mesh,
        compiler_params=pltpu.CompilerParams(needs_layout_passes=False),
        scratch_types=[
            pltpu.VMEM((2, EB), jnp.int32),
            pltpu.VMEM((2, EB), jnp.int32),
            pltpu.VMEM((EB, HID), jnp.float32),
            pltpu.VMEM((EB, HID), jnp.float32),
            pltpu.VMEM((EB, 16), jnp.float32),
            pltpu.SemaphoreType.DMA,
            pltpu.SemaphoreType.DMA,
        ],
    )
    def edge_logits(dst_hbm, src_hbm, u1_hbm, u2_hbm, xw_hbm,
                    w1_hbm, w2_hbm,
                    aggv, othv, urows, xrows, wrow, sem1, sem2):
        cid = lax.axis_index("c")
        sid = lax.axis_index("s")
        wid = sid * NC + cid
        iota = lax.broadcasted_iota(jnp.int32, (16,), 0)
        ebase0 = wid * E_PER_TILE

        def run_conv(agg_hbm, oth_hbm, u_hbm, w_hbm):
            def batch_body(t, carry):
                base = ebase0 + t * EB
                pltpu.sync_copy(agg_hbm.at[pl.ds(base, EB)], aggv.at[0])
                pltpu.sync_copy(oth_hbm.at[pl.ds(base, EB)], othv.at[0])
                cp1 = pltpu.async_copy(u_hbm.at[aggv.at[0]], urows, sem1)
                cp2 = pltpu.async_copy(xw_hbm.at[othv.at[0]], xrows, sem2)
                cp1.wait()
                cp2.wait()

                def edge_body(e, carry):
                    p = urows[e, pl.ds(0, 16)] * xrows[e, pl.ds(0, 16)]
                    for c in range(1, HID // 16):
                        p = p + (urows[e, pl.ds(c * 16, 16)]
                                 * xrows[e, pl.ds(c * 16, 16)])
                    a = jnp.sum(p)
                    wv = jnp.exp(jnp.full((16,), a, jnp.float32))
                    wrow[e, pl.ds(0, 16)] = jnp.where(iota == 0, wv, 0.0)
                    return carry
                lax.fori_loop(0, EB, edge_body, 0)

                pltpu.sync_copy(wrow, w_hbm.at[pl.ds(base, EB)])
                return carry

            lax.fori_loop(0, N_BATCH, batch_body, 0)

        run_conv(dst_hbm, src_hbm, u1_hbm, w1_hbm)
        run_conv(src_hbm, dst_hbm, u2_hbm, w2_hbm)

    return edge_logits


_make_edge_logits = functools.lru_cache(maxsize=None)(_make_edge_logits)


def _finish_body(m_ref, w_ref, W_ref, b_ref, o_ref):
    y = m_ref[...] / (w_ref[...][:, 0:1] + 1e-6)
    o_ref[...] = jnp.tanh(
        jnp.dot(y, W_ref[...].T, preferred_element_type=jnp.float32)
        + b_ref[...])


def _finish(outm, outw, nseg, w, b, blk):
    return pl.pallas_call(
        _finish_body,
        grid=(nseg // blk,),
        in_specs=[
            pl.BlockSpec((blk, HID), lambda i: (i, 0)),
            pl.BlockSpec((blk, 1), lambda i: (i, 0)),
            pl.BlockSpec((HID, HID), lambda i: (0, 0)),
            pl.BlockSpec((1, HID), lambda i: (0, 0)),
        ],
        out_specs=pl.BlockSpec((blk, HID), lambda i: (i, 0)),
        out_shape=jax.ShapeDtypeStruct((nseg, HID), jnp.float32),
    )(outm, outw, w, b.reshape(1, HID))


def kernel(x, edge_index, num_main, wam_w, wam_b, was_w, was_b,
           cm_supe_w, cm_supe_b, cm_attn_w, cm_attn_b, cm_main_w, cm_main_b,
           cs_supe_w, cs_supe_b, cs_attn_w, cs_attn_b, cs_main_w, cs_main_b,
           m2s_w, m2s_b, s2m_w, s2m_b):
    src = edge_index[0]
    dst = edge_index[1]
    xw, u1, m1, u2, m2 = _node_stage(
        x, wam_w, wam_b, was_w, was_b,
        cm_supe_w, cm_supe_b, cm_attn_w, cm_main_w, cm_main_b,
        cs_supe_w, cs_supe_b, cs_attn_w, cs_main_w, cs_main_b)
    w1o, w2o = _make_edge_logits()(dst, src, u1, u2, xw)
    w1 = w1o[:, 0]
    w2 = w2o[:, 0]
    keep1 = dst >= N_MAIN
    seg1 = jnp.where(keep1, dst - N_MAIN, 0)
    w1k = jnp.where(keep1, w1, 0.0)
    s1 = jax.ops.segment_sum(w1k, seg1, num_segments=N_SUPE)
    acc1 = jax.ops.segment_sum(w1k[:, None] * m1[src], seg1,
                               num_segments=N_SUPE)
    keep2 = src < N_MAIN
    seg2 = jnp.where(keep2, src, 0)
    w2k = jnp.where(keep2, w2, 0.0)
    s2 = jax.ops.segment_sum(w2k, seg2, num_segments=N_MAIN)
    acc2 = jax.ops.segment_sum(w2k[:, None] * m2[dst], seg2,
                               num_segments=N_MAIN)
    main_to_supe = _finish(acc1, s1[:, None], N_SUPE, m2s_w, m2s_b, 400)
    supe_to_main = _finish(acc2, s2[:, None], N_MAIN, s2m_w, s2m_b, 1000)
    return (supe_to_main, main_to_supe)


# speedup vs baseline: 1.2087x; 1.2087x over previous
import functools

import jax
import jax.numpy as jnp
from jax import lax
from jax.experimental import pallas as pl
from jax.experimental.pallas import tpu as pltpu
from jax.experimental.pallas import tpu_sc as plsc

HID = 128
N_NODES = 10000
N_MAIN = 8000
N_SUPE = N_NODES - N_MAIN
N_EDGES = 320000
ROW_BLK = 1000

NC = 2
NS = 16
NW = NC * NS
EB = 80


def _node_stage_body(x_ref, wam_ref, wamb_ref, was_ref, wasb_ref,
                     cmsw_ref, cmsb_ref, cmaw_ref, cmmw_ref, cmmb_ref,
                     cssw_ref, cssb_ref, csaw_ref, csmw_ref, csmb_ref,
                     xw_ref, u1_ref, m1_ref, u2_ref, m2_ref):
    i = pl.program_id(0)
    is_main = i * ROW_BLK < N_MAIN
    xb = x_ref[...]
    w = jnp.where(is_main, wam_ref[...], was_ref[...])
    b = jnp.where(is_main, wamb_ref[...], wasb_ref[...])
    xw = jnp.tanh(jnp.dot(xb, w.T, preferred_element_type=jnp.float32) + b)
    xw_ref[...] = xw
    u1_ref[...] = (jnp.dot(xw, cmsw_ref[...].T,
                           preferred_element_type=jnp.float32)
                   + cmsb_ref[...]) * cmaw_ref[...]
    m1_ref[...] = jnp.dot(xw, cmmw_ref[...].T,
                          preferred_element_type=jnp.float32) + cmmb_ref[...]
    u2_ref[...] = (jnp.dot(xw, cssw_ref[...].T,
                           preferred_element_type=jnp.float32)
                   + cssb_ref[...]) * csaw_ref[...]
    m2_ref[...] = jnp.dot(xw, csmw_ref[...].T,
                          preferred_element_type=jnp.float32) + csmb_ref[...]


def _node_stage(x, wam_w, wam_b, was_w, was_b,
                cm_supe_w, cm_supe_b, cm_attn_w, cm_main_w, cm_main_b,
                cs_supe_w, cs_supe_b, cs_attn_w, cs_main_w, cs_main_b):
    n = x.shape[0]
    row_spec = pl.BlockSpec((ROW_BLK, HID), lambda i: (i, 0))
    full_spec = pl.BlockSpec((HID, HID), lambda i: (0, 0))
    vec_spec = pl.BlockSpec((1, HID), lambda i: (0, 0))
    out_sds = jax.ShapeDtypeStruct((n, HID), jnp.float32)
    return pl.pallas_call(
        _node_stage_body,
        grid=(n // ROW_BLK,),
        in_specs=[row_spec,
                  full_spec, vec_spec, full_spec, vec_spec,
                  full_spec, vec_spec, vec_spec, full_spec, vec_spec,
                  full_spec, vec_spec, vec_spec, full_spec, vec_spec],
        out_specs=[row_spec] * 5,
        out_shape=[out_sds] * 5,
    )(x, wam_w, wam_b.reshape(1, HID), was_w, was_b.reshape(1, HID),
      cm_supe_w, cm_supe_b.reshape(1, HID), cm_attn_w, cm_main_w,
      cm_main_b.reshape(1, HID),
      cs_supe_w, cs_supe_b.reshape(1, HID), cs_attn_w, cs_main_w,
      cs_main_b.reshape(1, HID))


E_PER_TILE = N_EDGES // NW
N_BATCH = E_PER_TILE // EB


def _make_edge_logits():
    mesh = plsc.VectorSubcoreMesh(core_axis_name="c", subcore_axis_name="s",
                                  num_cores=NC, num_subcores=NS)

    @functools.partial(
        pl.kernel,
        out_type=[jax.ShapeDtypeStruct((N_EDGES, 16), jnp.float32),
                  jax.ShapeDtypeStruct((N_EDGES, 16), jnp.float32),
                  jax.ShapeDtypeStruct((N_EDGES, HID), jnp.float32),
                  jax.ShapeDtypeStruct((N_EDGES, HID), jnp.float32)],
        mesh=mesh,
        compiler_params=pltpu.CompilerParams(needs_layout_passes=False),
        scratch_types=[
            pltpu.VMEM((2, EB), jnp.int32),
            pltpu.VMEM((2, EB), jnp.int32),
            pltpu.VMEM((EB, HID), jnp.float32),
            pltpu.VMEM((EB, HID), jnp.float32),
            pltpu.VMEM((EB, 16), jnp.float32),
            pltpu.VMEM((EB, HID), jnp.float32),
            pltpu.VMEM((EB, HID), jnp.float32),
            pltpu.SemaphoreType.DMA,
            pltpu.SemaphoreType.DMA,
            pltpu.SemaphoreType.DMA,
        ],
    )
    def edge_logits(dst_hbm, src_hbm, u1_hbm, u2_hbm, m1_hbm, m2_hbm,
                    xw_hbm, w1_hbm, w2_hbm, msg1_hbm, msg2_hbm,
                    aggv, othv, urows, xrows, wrow, mrows, msg,
                    sem1, sem2, sem3):
        cid = lax.axis_index("c")
        sid = lax.axis_index("s")
        wid = sid * NC + cid
        iota = lax.broadcasted_iota(jnp.int32, (16,), 0)
        ebase0 = wid * E_PER_TILE

        def run_conv(agg_hbm, oth_hbm, u_hbm, m_hbm, w_hbm, msg_hbm):
            def batch_body(t, carry):
                base = ebase0 + t * EB
                pltpu.sync_copy(agg_hbm.at[pl.ds(base, EB)], aggv.at[0])
                pltpu.sync_copy(oth_hbm.at[pl.ds(base, EB)], othv.at[0])
                cp1 = pltpu.async_copy(u_hbm.at[aggv.at[0]], urows, sem1)
                cp2 = pltpu.async_copy(xw_hbm.at[othv.at[0]], xrows, sem2)
                cp3 = pltpu.async_copy(m_hbm.at[othv.at[0]], mrows, sem3)
                cp1.wait()
                cp2.wait()
                cp3.wait()

                def edge_body(e, carry):
                    p = urows[e, pl.ds(0, 16)] * xrows[e, pl.ds(0, 16)]
                    for c in range(1, HID // 16):
                        p = p + (urows[e, pl.ds(c * 16, 16)]
                                 * xrows[e, pl.ds(c * 16, 16)])
                    a = jnp.sum(p)
                    wv = jnp.exp(jnp.full((16,), a, jnp.float32))
                    wrow[e, pl.ds(0, 16)] = jnp.where(iota == 0, wv, 0.0)
                    for c in range(HID // 16):
                        msg[e, pl.ds(c * 16, 16)] = (
                            mrows[e, pl.ds(c * 16, 16)] * wv)
                    return carry
                lax.fori_loop(0, EB, edge_body, 0)

                pltpu.sync_copy(wrow, w_hbm.at[pl.ds(base, EB)])
                pltpu.sync_copy(msg, msg_hbm.at[pl.ds(base, EB)])
                return carry

            lax.fori_loop(0, N_BATCH, batch_body, 0)

        run_conv(dst_hbm, src_hbm, u1_hbm, m1_hbm, w1_hbm, msg1_hbm)
        run_conv(src_hbm, dst_hbm, u2_hbm, m2_hbm, w2_hbm, msg2_hbm)

    return edge_logits


_make_edge_logits = functools.lru_cache(maxsize=None)(_make_edge_logits)


def _finish_body(m_ref, w_ref, W_ref, b_ref, o_ref):
    y = m_ref[...] / (w_ref[...][:, 0:1] + 1e-6)
    o_ref[...] = jnp.tanh(
        jnp.dot(y, W_ref[...].T, preferred_element_type=jnp.float32)
        + b_ref[...])


def _finish(outm, outw, nseg, w, b, blk):
    return pl.pallas_call(
        _finish_body,
        grid=(nseg // blk,),
        in_specs=[
            pl.BlockSpec((blk, HID), lambda i: (i, 0)),
            pl.BlockSpec((blk, 1), lambda i: (i, 0)),
            pl.BlockSpec((HID, HID), lambda i: (0, 0)),
            pl.BlockSpec((1, HID), lambda i: (0, 0)),
        ],
        out_specs=pl.BlockSpec((blk, HID), lambda i: (i, 0)),
        out_shape=jax.ShapeDtypeStruct((nseg, HID), jnp.float32),
    )(outm, outw, w, b.reshape(1, HID))


def kernel(x, edge_index, num_main, wam_w, wam_b, was_w, was_b,
           cm_supe_w, cm_supe_b, cm_attn_w, cm_attn_b, cm_main_w, cm_main_b,
           cs_supe_w, cs_supe_b, cs_attn_w, cs_attn_b, cs_main_w, cs_main_b,
           m2s_w, m2s_b, s2m_w, s2m_b):
    src = edge_index[0]
    dst = edge_index[1]
    xw, u1, m1, u2, m2 = _node_stage(
        x, wam_w, wam_b, was_w, was_b,
        cm_supe_w, cm_supe_b, cm_attn_w, cm_main_w, cm_main_b,
        cs_supe_w, cs_supe_b, cs_attn_w, cs_main_w, cs_main_b)
    w1o, w2o, msg1, msg2 = _make_edge_logits()(dst, src, u1, u2, m1, m2, xw)
    w1 = w1o[:, 0]
    w2 = w2o[:, 0]
    keep1 = dst >= N_MAIN
    seg1 = jnp.where(keep1, dst - N_MAIN, N_SUPE)
    s1 = jax.ops.segment_sum(jnp.where(keep1, w1, 0.0), seg1,
                             num_segments=N_SUPE + 8)
    acc1 = jax.ops.segment_sum(msg1, seg1, num_segments=N_SUPE + 8)
    keep2 = src < N_MAIN
    seg2 = jnp.where(keep2, src, N_MAIN)
    s2 = jax.ops.segment_sum(jnp.where(keep2, w2, 0.0), seg2,
                             num_segments=N_MAIN + 8)
    acc2 = jax.ops.segment_sum(msg2, seg2, num_segments=N_MAIN + 8)
    main_to_supe = _finish(acc1[:N_SUPE], s1[:N_SUPE, None], N_SUPE,
                           m2s_w, m2s_b, 400)
    supe_to_main = _finish(acc2[:N_MAIN], s2[:N_MAIN, None], N_MAIN,
                           s2m_w, s2m_b, 1000)
    return (supe_to_main, main_to_supe)
